# Initial kernel scaffold; baseline (speedup 1.0000x reference)
import functools

import jax
import jax.numpy as jnp
from jax import lax
from jax.experimental import pallas as pl
from jax.experimental.pallas import tpu as pltpu
from jax.experimental.pallas import tpu_sc as plsc

NC = 2
NS = 16
NW = NC * NS
LANES = 16
D = 128
DB = D // LANES

CHUNK = 80
STRIPE_CHUNK = 80


def _dense2_tc(xp, Wl, bl, Wr, br):
  NP = xp.shape[0]
  blk = NP // 8

  def body(x_ref, wl_ref, bl_ref, wr_ref, br_ref, xl_ref, xr_ref):
    xb = x_ref[...]
    dn = (((1,), (1,)), ((), ()))
    xl_ref[...] = lax.dot_general(
        xb, wl_ref[...], dn, preferred_element_type=jnp.float32) + bl_ref[...]
    xr_ref[...] = lax.dot_general(
        xb, wr_ref[...], dn, preferred_element_type=jnp.float32) + br_ref[...]

  return pl.pallas_call(
      body,
      grid=(8,),
      in_specs=[
          pl.BlockSpec((blk, D), lambda i: (i, 0)),
          pl.BlockSpec((D, D), lambda i: (0, 0)),
          pl.BlockSpec((1, D), lambda i: (0, 0)),
          pl.BlockSpec((D, D), lambda i: (0, 0)),
          pl.BlockSpec((1, D), lambda i: (0, 0)),
      ],
      out_specs=[
          pl.BlockSpec((blk, D), lambda i: (i, 0)),
          pl.BlockSpec((blk, D), lambda i: (i, 0)),
      ],
      out_shape=[
          jax.ShapeDtypeStruct((NP, D), jnp.float32),
          jax.ShapeDtypeStruct((NP, D), jnp.float32),
      ],
  )(xp, Wl.reshape(D, D), bl.reshape(1, D), Wr.reshape(D, D),
    br.reshape(1, D))


def _mid_tc(acc, den, bias1, W2l, b2l, W2r, b2r):
  NP = acc.shape[1]
  blk = NP // 8

  def body(acc_ref, den_ref, b1_ref, wl_ref, bl_ref, wr_ref, br_ref,
           xl_ref, xr_ref):
    dsum = jnp.sum(den_ref[...], axis=0)
    asum = acc_ref[0] + acc_ref[1]
    h = asum / (dsum + 1e-16)[:, None] + b1_ref[...]
    h = jnp.maximum(h, 0.0)
    dn = (((1,), (1,)), ((), ()))
    xl_ref[...] = lax.dot_general(
        h, wl_ref[...], dn, preferred_element_type=jnp.float32) + bl_ref[...]
    xr_ref[...] = lax.dot_general(
        h, wr_ref[...], dn, preferred_element_type=jnp.float32) + br_ref[...]

  return pl.pallas_call(
      body,
      grid=(8,),
      in_specs=[
          pl.BlockSpec((NC, blk, D), lambda i: (0, i, 0)),
          pl.BlockSpec((NW, blk), lambda i: (0, i)),
          pl.BlockSpec((1, D), lambda i: (0, 0)),
          pl.BlockSpec((D, D), lambda i: (0, 0)),
          pl.BlockSpec((1, D), lambda i: (0, 0)),
          pl.BlockSpec((D, D), lambda i: (0, 0)),
          pl.BlockSpec((1, D), lambda i: (0, 0)),
      ],
      out_specs=[
          pl.BlockSpec((blk, D), lambda i: (i, 0)),
          pl.BlockSpec((blk, D), lambda i: (i, 0)),
      ],
      out_shape=[
          jax.ShapeDtypeStruct((NP, D), jnp.float32),
          jax.ShapeDtypeStruct((NP, D), jnp.float32),
      ],
  )(acc, den, bias1.reshape(1, D), W2l.reshape(D, D), b2l.reshape(1, D),
    W2r.reshape(D, D), b2r.reshape(1, D))


def _final_tc(acc, den, bias):
  NP = acc.shape[1]
  blk = NP // 8

  def body(acc_ref, den_ref, b_ref, out_ref):
    dsum = jnp.sum(den_ref[...], axis=0)
    asum = acc_ref[0] + acc_ref[1]
    out_ref[...] = asum / (dsum + 1e-16)[:, None] + b_ref[...]

  return pl.pallas_call(
      body,
      grid=(8,),
      in_specs=[
          pl.BlockSpec((NC, blk, D), lambda i: (0, i, 0)),
          pl.BlockSpec((NW, blk), lambda i: (0, i)),
          pl.BlockSpec((1, D), lambda i: (0, 0)),
      ],
      out_specs=pl.BlockSpec((blk, D), lambda i: (i, 0)),
      out_shape=jax.ShapeDtypeStruct((NP, D), jnp.float32),
  )(acc, den, bias.reshape(1, D))


def _edge_pass_sc(xl, xr, att, src, dst, NP, TPW):
  groups = TPW // CHUNK
  stripe = NP // NS

  mesh = plsc.VectorSubcoreMesh(
      core_axis_name="c", subcore_axis_name="s", num_cores=NC,
      num_subcores=NS)

  @functools.partial(
      pl.kernel,
      out_type=[
          jax.ShapeDtypeStruct((NC, NP, D), jnp.float32),
          jax.ShapeDtypeStruct((NW, NP), jnp.float32),
      ],
      mesh=mesh,
      scratch_types=[
          pltpu.VMEM((2, CHUNK), jnp.int32),
          pltpu.VMEM((2, CHUNK), jnp.int32),
          pltpu.VMEM((CHUNK, D), jnp.float32),
          pltpu.VMEM((CHUNK, D), jnp.float32),
          pltpu.VMEM((NP,), jnp.float32),
          pltpu.VMEM((D,), jnp.float32),
          pltpu.VMEM_SHARED((NP, D), jnp.float32),
          pltpu.SemaphoreType.DMA,
          pltpu.SemaphoreType.DMA,
      ],
  )
  def k(xl_hbm, xr_hbm, att_hbm, src_hbm, dst_hbm, acc_out, den_out,
        sidx, didx, rows_s, rows_d, den_t, att_v, acc_sh, sem_s, sem_d):
    cid = lax.axis_index("c")
    sid = lax.axis_index("s")
    wid = sid * NC + cid
    base = wid * TPW

    pltpu.sync_copy(att_hbm, att_v)
    att_vecs = [att_v[pl.ds(b * LANES, LANES)] for b in range(DB)]
    lane0 = lax.iota(jnp.int32, LANES) == 0
    zv = jnp.zeros((LANES,), jnp.float32)

    def zden(j, _):
      den_t[pl.ds(j * LANES, LANES)] = zv
      return 0
    lax.fori_loop(0, NP // LANES, zden, 0)

    def zrows(j, _):
      rows_s[j // DB, pl.ds((j % DB) * LANES, LANES)] = zv
      return 0
    lax.fori_loop(0, CHUNK * DB, zrows, 0)
    for j in range(stripe // STRIPE_CHUNK):
      pltpu.sync_copy(
          rows_s,
          acc_sh.at[pl.ds(sid * stripe + j * STRIPE_CHUNK, STRIPE_CHUNK)])
    plsc.subcore_barrier()

    def group(g, _):
      off = base + g * CHUNK
      pltpu.sync_copy(src_hbm.at[pl.ds(off, CHUNK)], sidx.at[0])
      pltpu.sync_copy(dst_hbm.at[pl.ds(off, CHUNK)], didx.at[0])
      cp_s = pltpu.async_copy(xl_hbm.at[sidx.at[0]], rows_s, sem_s)
      cp_d = pltpu.async_copy(xr_hbm.at[didx.at[0]], rows_d, sem_d)
      cp_s.wait()
      cp_d.wait()

      def edge(i, _):
        accv = jnp.zeros((LANES,), jnp.float32)
        for b in range(DB):
          sv = rows_s[i, pl.ds(b * LANES, LANES)]
          dv = rows_d[i, pl.ds(b * LANES, LANES)]
          v = sv + dv
          lr = jnp.maximum(v, v * 0.2)
          accv = accv + att_vecs[b] * lr
        e = jnp.sum(accv)
        pv = jnp.exp(jnp.full((LANES,), e))
        for b in range(DB):
          rows_s[i, pl.ds(b * LANES, LANES)] = (
              rows_s[i, pl.ds(b * LANES, LANES)] * pv)
        dsc = didx[0, i]
        plsc.addupdate_scatter(
            den_t, [jnp.full((LANES,), dsc)], pv, mask=lane0)
        return 0

      lax.fori_loop(0, CHUNK, edge, 0)
      pltpu.sync_copy(rows_s, acc_sh.at[didx.at[0]], add=True)
      return 0

    lax.fori_loop(0, groups, group, 0)
    plsc.subcore_barrier()

    pltpu.sync_copy(acc_sh.at[pl.ds(sid * stripe, stripe)],
                    acc_out.at[cid, pl.ds(sid * stripe, stripe)])
    pltpu.sync_copy(den_t, den_out.at[wid])

  return k(xl, xr, att, src, dst)


def kernel(x, edge_index, W1l, b1l, W1r, b1r, att1, bias1,
           W2l, b2l, W2r, b2r, att2, bias2):
  N = x.shape[0]
  E = edge_index.shape[1]

  NP = -(-N // (NS * STRIPE_CHUNK)) * (NS * STRIPE_CHUNK)

  EL = E + N
  TPW = -(-EL // (NW * CHUNK)) * CHUNK
  EP = TPW * NW

  loop = jnp.arange(N, dtype=jnp.int32)
  padi = jnp.full((EP - EL,), N, jnp.int32)
  src = jnp.concatenate([edge_index[0], loop, padi])
  dst = jnp.concatenate([edge_index[1], loop, padi])

  xp = jnp.zeros((NP, D), jnp.float32).at[:N].set(x)

  xl1, xr1 = _dense2_tc(xp, W1l, b1l, W1r, b1r)
  acc1, den1 = _edge_pass_sc(xl1, xr1, att1, src, dst, NP, TPW)
  xl2, xr2 = _mid_tc(acc1, den1, bias1, W2l, b2l, W2r, b2r)
  acc2, den2 = _edge_pass_sc(xl2, xr2, att2, src, dst, NP, TPW)
  out = _final_tc(acc2, den2, bias2)
  return out[:N]


# speedup vs baseline: 13.5631x; 13.5631x over previous
import functools

import jax
import jax.numpy as jnp
from jax import lax
from jax.experimental import pallas as pl
from jax.experimental.pallas import tpu as pltpu
from jax.experimental.pallas import tpu_sc as plsc

NC = 2
NS = 16
NW = NC * NS
LANES = 16
D = 128
DB = D // LANES

CHUNK = 80
STRIPE_CHUNK = 80


def _dense2_tc(xp, Wl, bl, Wr, br):
  NP = xp.shape[0]
  blk = NP // 8

  def body(x_ref, wl_ref, bl_ref, wr_ref, br_ref, xl_ref, xr_ref):
    xb = x_ref[...]
    dn = (((1,), (1,)), ((), ()))
    xl_ref[...] = lax.dot_general(
        xb, wl_ref[...], dn, preferred_element_type=jnp.float32) + bl_ref[...]
    xr_ref[...] = lax.dot_general(
        xb, wr_ref[...], dn, preferred_element_type=jnp.float32) + br_ref[...]

  return pl.pallas_call(
      body,
      grid=(8,),
      in_specs=[
          pl.BlockSpec((blk, D), lambda i: (i, 0)),
          pl.BlockSpec((D, D), lambda i: (0, 0)),
          pl.BlockSpec((1, D), lambda i: (0, 0)),
          pl.BlockSpec((D, D), lambda i: (0, 0)),
          pl.BlockSpec((1, D), lambda i: (0, 0)),
      ],
      out_specs=[
          pl.BlockSpec((blk, D), lambda i: (i, 0)),
          pl.BlockSpec((blk, D), lambda i: (i, 0)),
      ],
      out_shape=[
          jax.ShapeDtypeStruct((NP, D), jnp.float32),
          jax.ShapeDtypeStruct((NP, D), jnp.float32),
      ],
  )(xp, Wl.reshape(D, D), bl.reshape(1, D), Wr.reshape(D, D),
    br.reshape(1, D))


def _mid_tc(acc, den, bias1, W2l, b2l, W2r, b2r):
  NP = acc.shape[1]
  blk = NP // 8

  def body(acc_ref, den_ref, b1_ref, wl_ref, bl_ref, wr_ref, br_ref,
           xl_ref, xr_ref):
    dsum = jnp.sum(den_ref[...], axis=0)
    asum = acc_ref[0] + acc_ref[1]
    h = asum / (dsum + 1e-16)[:, None] + b1_ref[...]
    h = jnp.maximum(h, 0.0)
    dn = (((1,), (1,)), ((), ()))
    xl_ref[...] = lax.dot_general(
        h, wl_ref[...], dn, preferred_element_type=jnp.float32) + bl_ref[...]
    xr_ref[...] = lax.dot_general(
        h, wr_ref[...], dn, preferred_element_type=jnp.float32) + br_ref[...]

  return pl.pallas_call(
      body,
      grid=(8,),
      in_specs=[
          pl.BlockSpec((NC, blk, D), lambda i: (0, i, 0)),
          pl.BlockSpec((NW, blk), lambda i: (0, i)),
          pl.BlockSpec((1, D), lambda i: (0, 0)),
          pl.BlockSpec((D, D), lambda i: (0, 0)),
          pl.BlockSpec((1, D), lambda i: (0, 0)),
          pl.BlockSpec((D, D), lambda i: (0, 0)),
          pl.BlockSpec((1, D), lambda i: (0, 0)),
      ],
      out_specs=[
          pl.BlockSpec((blk, D), lambda i: (i, 0)),
          pl.BlockSpec((blk, D), lambda i: (i, 0)),
      ],
      out_shape=[
          jax.ShapeDtypeStruct((NP, D), jnp.float32),
          jax.ShapeDtypeStruct((NP, D), jnp.float32),
      ],
  )(acc, den, bias1.reshape(1, D), W2l.reshape(D, D), b2l.reshape(1, D),
    W2r.reshape(D, D), b2r.reshape(1, D))


def _final_tc(acc, den, bias):
  NP = acc.shape[1]
  blk = NP // 8

  def body(acc_ref, den_ref, b_ref, out_ref):
    dsum = jnp.sum(den_ref[...], axis=0)
    asum = acc_ref[0] + acc_ref[1]
    out_ref[...] = asum / (dsum + 1e-16)[:, None] + b_ref[...]

  return pl.pallas_call(
      body,
      grid=(8,),
      in_specs=[
          pl.BlockSpec((NC, blk, D), lambda i: (0, i, 0)),
          pl.BlockSpec((NW, blk), lambda i: (0, i)),
          pl.BlockSpec((1, D), lambda i: (0, 0)),
      ],
      out_specs=pl.BlockSpec((blk, D), lambda i: (i, 0)),
      out_shape=jax.ShapeDtypeStruct((NP, D), jnp.float32),
  )(acc, den, bias.reshape(1, D))


def _edge_pass_sc(xl, xr, att, src, dst, NP, TPW):
  groups = TPW // CHUNK
  stripe = NP // NS

  mesh = plsc.VectorSubcoreMesh(
      core_axis_name="c", subcore_axis_name="s", num_cores=NC,
      num_subcores=NS)

  @functools.partial(
      pl.kernel,
      out_type=[
          jax.ShapeDtypeStruct((NC, NP, D), jnp.float32),
          jax.ShapeDtypeStruct((NW, NP), jnp.float32),
      ],
      mesh=mesh,
      compiler_params=pltpu.CompilerParams(needs_layout_passes=False),
      scratch_types=[
          pltpu.VMEM((2, CHUNK), jnp.int32),
          pltpu.VMEM((2, CHUNK), jnp.int32),
          pltpu.VMEM((CHUNK, D), jnp.float32),
          pltpu.VMEM((CHUNK, D), jnp.float32),
          pltpu.VMEM((NP,), jnp.float32),
          pltpu.VMEM((D,), jnp.float32),
          pltpu.VMEM_SHARED((NP, D), jnp.float32),
          pltpu.SemaphoreType.DMA,
          pltpu.SemaphoreType.DMA,
      ],
  )
  def k(xl_hbm, xr_hbm, att_hbm, src_hbm, dst_hbm, acc_out, den_out,
        sidx, didx, rows_s, rows_d, den_t, att_v, acc_sh, sem_s, sem_d):
    cid = lax.axis_index("c")
    sid = lax.axis_index("s")
    wid = sid * NC + cid
    base = wid * TPW

    pltpu.sync_copy(att_hbm, att_v)
    att_vecs = [att_v[pl.ds(b * LANES, LANES)] for b in range(DB)]
    lane_iota = lax.iota(jnp.int32, LANES)
    last_lane = jnp.full((LANES,), LANES - 1, jnp.int32)
    zv = jnp.zeros((LANES,), jnp.float32)

    def zden(j, _):
      den_t[pl.ds(j * LANES, LANES)] = zv
      return 0
    lax.fori_loop(0, NP // LANES, zden, 0)

    def zrows(j, _):
      rows_s[j // DB, pl.ds((j % DB) * LANES, LANES)] = zv
      return 0
    lax.fori_loop(0, CHUNK * DB, zrows, 0)
    for j in range(stripe // STRIPE_CHUNK):
      pltpu.sync_copy(
          rows_s,
          acc_sh.at[pl.ds(sid * stripe + j * STRIPE_CHUNK, STRIPE_CHUNK)])
    plsc.subcore_barrier()

    def group(g, _):
      off = base + g * CHUNK
      pltpu.sync_copy(src_hbm.at[pl.ds(off, CHUNK)], sidx.at[0])
      pltpu.sync_copy(dst_hbm.at[pl.ds(off, CHUNK)], didx.at[0])
      cp_s = pltpu.async_copy(xl_hbm.at[sidx.at[0]], rows_s, sem_s)
      cp_d = pltpu.async_copy(xr_hbm.at[didx.at[0]], rows_d, sem_d)
      cp_s.wait()
      cp_d.wait()

      def edge16(j, _):
        dvec = didx[0, pl.ds(j * LANES, LANES)]
        pvals = zv
        for ii in range(LANES):
          i = j * LANES + ii
          accv = jnp.zeros((LANES,), jnp.float32)
          for b in range(DB):
            sv = rows_s[i, pl.ds(b * LANES, LANES)]
            dv = rows_d[i, pl.ds(b * LANES, LANES)]
            v = sv + dv
            lr = jnp.maximum(v, v * 0.2)
            accv = accv + att_vecs[b] * lr
          cs = plsc.cumsum(accv)
          pv = jnp.exp(cs.at[last_lane].get(mode="promise_in_bounds"))
          for b in range(DB):
            rows_s[i, pl.ds(b * LANES, LANES)] = (
                rows_s[i, pl.ds(b * LANES, LANES)] * pv)
          pvals = jnp.where(lane_iota == ii, pv, pvals)
        plsc.addupdate_scatter(den_t, [dvec], pvals)
        return 0

      lax.fori_loop(0, CHUNK // LANES, edge16, 0)
      pltpu.sync_copy(rows_s, acc_sh.at[didx.at[0]], add=True)
      return 0

    lax.fori_loop(0, groups, group, 0)
    plsc.subcore_barrier()

    pltpu.sync_copy(acc_sh.at[pl.ds(sid * stripe, stripe)],
                    acc_out.at[cid, pl.ds(sid * stripe, stripe)])
    pltpu.sync_copy(den_t, den_out.at[wid])

  return k(xl, xr, att, src, dst)


def kernel(x, edge_index, W1l, b1l, W1r, b1r, att1, bias1,
           W2l, b2l, W2r, b2r, att2, bias2):
  N = x.shape[0]
  E = edge_index.shape[1]

  NP = -(-N // (NS * STRIPE_CHUNK)) * (NS * STRIPE_CHUNK)

  EL = E + N
  TPW = -(-EL // (NW * CHUNK)) * CHUNK
  EP = TPW * NW

  loop = jnp.arange(N, dtype=jnp.int32)
  padi = jnp.full((EP - EL,), N, jnp.int32)
  src = jnp.concatenate([edge_index[0], loop, padi])
  dst = jnp.concatenate([edge_index[1], loop, padi])

  xp = jnp.zeros((NP, D), jnp.float32).at[:N].set(x)

  xl1, xr1 = _dense2_tc(xp, W1l, b1l, W1r, b1r)
  acc1, den1 = _edge_pass_sc(xl1, xr1, att1, src, dst, NP, TPW)
  xl2, xr2 = _mid_tc(acc1, den1, bias1, W2l, b2l, W2r, b2r)
  acc2, den2 = _edge_pass_sc(xl2, xr2, att2, src, dst, NP, TPW)
  out = _final_tc(acc2, den2, bias2)
  return out[:N]
